# Initial kernel scaffold; baseline (speedup 1.0000x reference)
import jax
import jax.numpy as jnp
from jax.experimental import pallas as pl


def kernel(facts, mem0, W_emb, W_enc, W_wk, W_wv, W_rk):
    raise NotImplementedError("write your pallas kernel here")


# speedup vs baseline: 56.2886x; 56.2886x over previous
import functools

import jax
import jax.numpy as jnp
from jax import lax
from jax.experimental import pallas as pl
from jax.experimental.pallas import tpu as pltpu

S, B, I = 50, 256, 200
E, M = 80, 256
T = 50
T_TOT = S * T
A_I, A_V = 0.9, 0.9
ETA = 0.01
LAM = 0.951229424500714

WIN_LO, WIN_HI = 19, 48


def _spike(v):
    return jax.nn.sigmoid(10.0 * (v - 1.0))


def _dotT(x, w):
    return lax.dot_general(x, w, (((x.ndim - 1,), (1,)), ((), ())))


BLK_A = 64


def _chains_body(facts_ref, wemb_ref, wenc_ref, wwk_ref, wwv_ref, wrk_ref,
                 tk_out, zv_out, zrk_out,
                 ze_s, kin_s, vin_s, rin_s,
                 enc_i, enc_v, enc_z,
                 wk_i, wk_v, wk_z, wv_i, wv_v, wv_z, rk_i, rk_v, rk_z, tk_s):
    c = pl.program_id(1)

    @pl.when(c == 0)
    def _():
        for r in (enc_i, enc_v, enc_z, wk_i, wk_v, wk_z,
                  wv_i, wv_v, wv_z, rk_i, rk_v, rk_z, tk_s):
            r[...] = jnp.zeros_like(r)

    x = facts_ref[0]
    emb = _dotT(x, wemb_ref[...])
    enc_in = _dotT(emb, wenc_ref[...])

    def p1(j, _):
        i = A_I * enc_i[...] + enc_in
        v = A_V * enc_v[...] * (1.0 - enc_z[...]) + i
        z = _spike(v)
        enc_i[...] = i
        enc_v[...] = v
        enc_z[...] = z
        ze_s[pl.ds(j * BLK_A, BLK_A), :] = z
        return 0

    lax.fori_loop(0, T, p1, 0)

    ze = ze_s[...]
    kin_s[...] = _dotT(ze, wwk_ref[...])
    vin_s[...] = _dotT(ze, wwv_ref[...])
    rin_s[...] = _dotT(ze, wrk_ref[...])

    def p3(j, _):
        sl = pl.ds(j * BLK_A, BLK_A)
        ki = A_I * wk_i[...] + kin_s[sl, :]
        kv = A_V * wk_v[...] * (1.0 - wk_z[...]) + ki
        kz = _spike(kv)
        wk_i[...] = ki
        wk_v[...] = kv
        wk_z[...] = kz

        vi = A_I * wv_i[...] + vin_s[sl, :]
        vv = A_V * wv_v[...] * (1.0 - wv_z[...]) + vi
        vz = _spike(vv)
        wv_i[...] = vi
        wv_v[...] = vv
        wv_z[...] = vz

        ri = A_I * rk_i[...] + rin_s[sl, :]
        rv = A_V * rk_v[...] * (1.0 - rk_z[...]) + ri
        rz = _spike(rv)
        rk_i[...] = ri
        rk_v[...] = rv
        rk_z[...] = rz

        tkn = LAM * tk_s[...] + kz
        tk_s[...] = tkn

        tk_out[j] = tkn
        zv_out[j] = vz
        zrk_out[j] = rz
        return 0

    lax.fori_loop(0, T, p3, 0)


BLK_B = 8
CH_B = 125


def _memory_body(tk_ref, zv_ref, zrk_ref, mem0_ref, rv_out,
                 mem_s, tkT, zvT, zrkT, rvT):
    c = pl.program_id(1)

    @pl.when(c == 0)
    def _():
        mem_s[...] = mem0_ref[...]

    tkT[...] = jnp.swapaxes(tk_ref[...], 0, 1)
    zvT[...] = jnp.swapaxes(zv_ref[...], 0, 1)
    zrkT[...] = jnp.swapaxes(zrk_ref[...], 0, 1)

    row = lax.broadcasted_iota(jnp.int32, (CH_B, CH_B), 0)
    col = lax.broadcasted_iota(jnp.int32, (CH_B, CH_B), 1)
    causal = col <= row

    for be in range(BLK_B):
        tkb = tkT[be]
        zvb = zvT[be]
        zrkb = zrkT[be]
        memb = mem_s[be]
        g = lax.dot_general(zrkb, tkb, (((1,), (1,)), ((), ())))
        gm = jnp.where(causal, g, 0.0)
        intra = lax.dot_general(gm, zvb, (((1,), (0,)), ((), ())))
        inter = lax.dot_general(zrkb, memb, (((1,), (1,)), ((), ())))
        rvT[be] = inter + ETA * intra
        mem_s[be] = memb + ETA * lax.dot_general(
            zvb, tkb, (((0,), (0,)), ((), ())))

    rv_out[...] = jnp.swapaxes(rvT[...], 0, 1)


BLK_C = 128


def _readout_body(rv_ref, out_ref, ri_s, rv_s, rz_s, acc_s):
    c = pl.program_id(1)

    @pl.when(c == 0)
    def _():
        ri_s[...] = jnp.zeros_like(ri_s)
        rv_s[...] = jnp.zeros_like(rv_s)
        rz_s[...] = jnp.zeros_like(rz_s)

    acc_s[...] = jnp.zeros_like(acc_s)

    def step(j, carry, accumulate):
        i = A_I * ri_s[...] + rv_ref[j]
        v = A_V * rv_s[...] * (1.0 - rz_s[...]) + i
        z = _spike(v)
        ri_s[...] = i
        rv_s[...] = v
        rz_s[...] = z
        if accumulate:
            acc_s[...] = acc_s[...] + z
        return carry

    lax.fori_loop(0, WIN_LO, functools.partial(step, accumulate=False), 0)
    lax.fori_loop(WIN_LO, WIN_HI + 1, functools.partial(step, accumulate=True), 0)
    lax.fori_loop(WIN_HI + 1, T, functools.partial(step, accumulate=False), 0)

    out_ref[0] = acc_s[...]


def kernel(facts, mem0, W_emb, W_enc, W_wk, W_wv, W_rk):
    f32 = jnp.float32
    zseq = jax.ShapeDtypeStruct((T_TOT, B, M), f32)

    tk_all, zv_all, zrk_all = pl.pallas_call(
        _chains_body,
        out_shape=(zseq, zseq, zseq),
        grid=(B // BLK_A, S),
        in_specs=[
            pl.BlockSpec((1, BLK_A, I), lambda b, c: (c, b, 0)),
            pl.BlockSpec((E, I), lambda b, c: (0, 0)),
            pl.BlockSpec((E, E), lambda b, c: (0, 0)),
            pl.BlockSpec((M, E), lambda b, c: (0, 0)),
            pl.BlockSpec((M, E), lambda b, c: (0, 0)),
            pl.BlockSpec((M, E), lambda b, c: (0, 0)),
        ],
        out_specs=(
            pl.BlockSpec((T, BLK_A, M), lambda b, c: (c, b, 0)),
            pl.BlockSpec((T, BLK_A, M), lambda b, c: (c, b, 0)),
            pl.BlockSpec((T, BLK_A, M), lambda b, c: (c, b, 0)),
        ),
        scratch_shapes=[
            pltpu.VMEM((T * BLK_A, E), f32),
            pltpu.VMEM((T * BLK_A, M), f32),
            pltpu.VMEM((T * BLK_A, M), f32),
            pltpu.VMEM((T * BLK_A, M), f32),
        ] + [pltpu.VMEM((BLK_A, E), f32)] * 3
          + [pltpu.VMEM((BLK_A, M), f32)] * 10,
        compiler_params=pltpu.CompilerParams(
            dimension_semantics=("parallel", "arbitrary"),
            vmem_limit_bytes=100 * 1024 * 1024,
        ),
        name="lif_chains",
    )(facts, W_emb, W_enc, W_wk, W_wv, W_rk)

    rv_all = pl.pallas_call(
        _memory_body,
        out_shape=zseq,
        grid=(B // BLK_B, T_TOT // CH_B),
        in_specs=[
            pl.BlockSpec((CH_B, BLK_B, M), lambda b, c: (c, b, 0)),
            pl.BlockSpec((CH_B, BLK_B, M), lambda b, c: (c, b, 0)),
            pl.BlockSpec((CH_B, BLK_B, M), lambda b, c: (c, b, 0)),
            pl.BlockSpec((BLK_B, M, M), lambda b, c: (b, 0, 0)),
        ],
        out_specs=pl.BlockSpec((CH_B, BLK_B, M), lambda b, c: (c, b, 0)),
        scratch_shapes=[
            pltpu.VMEM((BLK_B, M, M), f32),
            pltpu.VMEM((BLK_B, CH_B, M), f32),
            pltpu.VMEM((BLK_B, CH_B, M), f32),
            pltpu.VMEM((BLK_B, CH_B, M), f32),
            pltpu.VMEM((BLK_B, CH_B, M), f32),
        ],
        compiler_params=pltpu.CompilerParams(
            dimension_semantics=("parallel", "arbitrary"),
            vmem_limit_bytes=100 * 1024 * 1024,
        ),
        name="memory_read",
    )(tk_all, zv_all, zrk_all, mem0)

    out = pl.pallas_call(
        _readout_body,
        out_shape=jax.ShapeDtypeStruct((S, B, M), f32),
        grid=(B // BLK_C, S),
        in_specs=[pl.BlockSpec((T, BLK_C, M), lambda b, c: (c, b, 0))],
        out_specs=pl.BlockSpec((1, BLK_C, M), lambda b, c: (c, b, 0)),
        scratch_shapes=[pltpu.VMEM((BLK_C, M), f32)] * 4,
        compiler_params=pltpu.CompilerParams(
            dimension_semantics=("parallel", "arbitrary"),
            vmem_limit_bytes=100 * 1024 * 1024,
        ),
        name="readout",
    )(rv_all)

    return out


# speedup vs baseline: 62.1107x; 1.1034x over previous
import functools

import jax
import jax.numpy as jnp
from jax import lax
from jax.experimental import pallas as pl
from jax.experimental.pallas import tpu as pltpu

S, B, I = 50, 256, 200
E, M = 80, 256
T = 50
T_TOT = S * T
A_I, A_V = 0.9, 0.9
ETA = 0.01
LAM = 0.951229424500714

WIN_LO, WIN_HI = 19, 48


def _spike(v):
    return jax.nn.sigmoid(10.0 * (v - 1.0))


def _dotT(x, w):
    return lax.dot_general(x, w, (((x.ndim - 1,), (1,)), ((), ())))


BLK_A = 64


def _chains_body(facts_ref, wemb_ref, wenc_ref, wwk_ref, wwv_ref, wrk_ref,
                 tk_out, zv_out, zrk_out,
                 ze_s, kin_s, vin_s, rin_s,
                 enc_i, enc_v, enc_z,
                 wk_i, wk_v, wk_z, wv_i, wv_v, wv_z, rk_i, rk_v, rk_z, tk_s):
    c = pl.program_id(1)

    @pl.when(c == 0)
    def _():
        for r in (enc_i, enc_v, enc_z, wk_i, wk_v, wk_z,
                  wv_i, wv_v, wv_z, rk_i, rk_v, rk_z, tk_s):
            r[...] = jnp.zeros_like(r)

    x = facts_ref[0]
    emb = _dotT(x, wemb_ref[...])
    enc_in = _dotT(emb, wenc_ref[...])

    def p1(j, _):
        i = A_I * enc_i[...] + enc_in
        v = A_V * enc_v[...] * (1.0 - enc_z[...]) + i
        z = _spike(v)
        enc_i[...] = i
        enc_v[...] = v
        enc_z[...] = z
        ze_s[pl.ds(j * BLK_A, BLK_A), :] = z
        return 0

    lax.fori_loop(0, T, p1, 0)

    ze = ze_s[...]
    kin_s[...] = _dotT(ze, wwk_ref[...])
    vin_s[...] = _dotT(ze, wwv_ref[...])
    rin_s[...] = _dotT(ze, wrk_ref[...])

    def p3(j, _):
        sl = pl.ds(j * BLK_A, BLK_A)
        ki = A_I * wk_i[...] + kin_s[sl, :]
        kv = A_V * wk_v[...] * (1.0 - wk_z[...]) + ki
        kz = _spike(kv)
        wk_i[...] = ki
        wk_v[...] = kv
        wk_z[...] = kz

        vi = A_I * wv_i[...] + vin_s[sl, :]
        vv = A_V * wv_v[...] * (1.0 - wv_z[...]) + vi
        vz = _spike(vv)
        wv_i[...] = vi
        wv_v[...] = vv
        wv_z[...] = vz

        ri = A_I * rk_i[...] + rin_s[sl, :]
        rv = A_V * rk_v[...] * (1.0 - rk_z[...]) + ri
        rz = _spike(rv)
        rk_i[...] = ri
        rk_v[...] = rv
        rk_z[...] = rz

        tkn = LAM * tk_s[...] + kz
        tk_s[...] = tkn

        tk_out[j] = tkn.astype(jnp.bfloat16)
        zv_out[j] = vz.astype(jnp.bfloat16)
        zrk_out[j] = rz.astype(jnp.bfloat16)
        return 0

    lax.fori_loop(0, T, p3, 0)


BLK_B = 8
CH_B = 125


def _memory_body(tk_ref, zv_ref, zrk_ref, mem0_ref, rv_out,
                 mem_s, tkT, zvT, zrkT, rvT):
    c = pl.program_id(1)

    @pl.when(c == 0)
    def _():
        mem_s[...] = mem0_ref[...]

    tkT[...] = jnp.swapaxes(tk_ref[...], 0, 1)
    zvT[...] = jnp.swapaxes(zv_ref[...], 0, 1)
    zrkT[...] = jnp.swapaxes(zrk_ref[...], 0, 1)

    row = lax.broadcasted_iota(jnp.int32, (CH_B, CH_B), 0)
    col = lax.broadcasted_iota(jnp.int32, (CH_B, CH_B), 1)
    causal = col <= row

    for be in range(BLK_B):
        tkb = tkT[be]
        zvb = zvT[be]
        zrkb = zrkT[be]
        memb = mem_s[be]
        g = lax.dot_general(zrkb, tkb, (((1,), (1,)), ((), ())),
                            preferred_element_type=jnp.float32)
        gm = jnp.where(causal, g, 0.0).astype(jnp.bfloat16)
        intra = lax.dot_general(gm, zvb, (((1,), (0,)), ((), ())),
                                preferred_element_type=jnp.float32)
        inter = lax.dot_general(zrkb, memb.astype(jnp.bfloat16),
                                (((1,), (1,)), ((), ())),
                                preferred_element_type=jnp.float32)
        rvT[be] = (inter + ETA * intra).astype(jnp.bfloat16)
        mem_s[be] = memb + ETA * lax.dot_general(
            zvb, tkb, (((0,), (0,)), ((), ())),
            preferred_element_type=jnp.float32)

    rv_out[...] = jnp.swapaxes(rvT[...], 0, 1)


BLK_C = 128


def _readout_body(rv_ref, out_ref, ri_s, rv_s, rz_s, acc_s):
    c = pl.program_id(1)

    @pl.when(c == 0)
    def _():
        ri_s[...] = jnp.zeros_like(ri_s)
        rv_s[...] = jnp.zeros_like(rv_s)
        rz_s[...] = jnp.zeros_like(rz_s)

    acc_s[...] = jnp.zeros_like(acc_s)

    def step(j, carry, accumulate):
        i = A_I * ri_s[...] + rv_ref[j]
        v = A_V * rv_s[...] * (1.0 - rz_s[...]) + i
        z = _spike(v)
        ri_s[...] = i
        rv_s[...] = v
        rz_s[...] = z
        if accumulate:
            acc_s[...] = acc_s[...] + z
        return carry

    lax.fori_loop(0, WIN_LO, functools.partial(step, accumulate=False), 0)
    lax.fori_loop(WIN_LO, WIN_HI + 1, functools.partial(step, accumulate=True), 0)
    lax.fori_loop(WIN_HI + 1, T, functools.partial(step, accumulate=False), 0)

    out_ref[0] = acc_s[...]


def kernel(facts, mem0, W_emb, W_enc, W_wk, W_wv, W_rk):
    f32 = jnp.float32
    bf16 = jnp.bfloat16
    zseq = jax.ShapeDtypeStruct((T_TOT, B, M), bf16)

    tk_all, zv_all, zrk_all = pl.pallas_call(
        _chains_body,
        out_shape=(zseq, zseq, zseq),
        grid=(B // BLK_A, S),
        in_specs=[
            pl.BlockSpec((1, BLK_A, I), lambda b, c: (c, b, 0)),
            pl.BlockSpec((E, I), lambda b, c: (0, 0)),
            pl.BlockSpec((E, E), lambda b, c: (0, 0)),
            pl.BlockSpec((M, E), lambda b, c: (0, 0)),
            pl.BlockSpec((M, E), lambda b, c: (0, 0)),
            pl.BlockSpec((M, E), lambda b, c: (0, 0)),
        ],
        out_specs=(
            pl.BlockSpec((T, BLK_A, M), lambda b, c: (c, b, 0)),
            pl.BlockSpec((T, BLK_A, M), lambda b, c: (c, b, 0)),
            pl.BlockSpec((T, BLK_A, M), lambda b, c: (c, b, 0)),
        ),
        scratch_shapes=[
            pltpu.VMEM((T * BLK_A, E), f32),
            pltpu.VMEM((T * BLK_A, M), f32),
            pltpu.VMEM((T * BLK_A, M), f32),
            pltpu.VMEM((T * BLK_A, M), f32),
        ] + [pltpu.VMEM((BLK_A, E), f32)] * 3
          + [pltpu.VMEM((BLK_A, M), f32)] * 10,
        compiler_params=pltpu.CompilerParams(
            dimension_semantics=("parallel", "arbitrary"),
            vmem_limit_bytes=100 * 1024 * 1024,
        ),
        name="lif_chains",
    )(facts, W_emb, W_enc, W_wk, W_wv, W_rk)

    rv_all = pl.pallas_call(
        _memory_body,
        out_shape=zseq,
        grid=(B // BLK_B, T_TOT // CH_B),
        in_specs=[
            pl.BlockSpec((CH_B, BLK_B, M), lambda b, c: (c, b, 0)),
            pl.BlockSpec((CH_B, BLK_B, M), lambda b, c: (c, b, 0)),
            pl.BlockSpec((CH_B, BLK_B, M), lambda b, c: (c, b, 0)),
            pl.BlockSpec((BLK_B, M, M), lambda b, c: (b, 0, 0)),
        ],
        out_specs=pl.BlockSpec((CH_B, BLK_B, M), lambda b, c: (c, b, 0)),
        scratch_shapes=[
            pltpu.VMEM((BLK_B, M, M), f32),
            pltpu.VMEM((BLK_B, CH_B, M), bf16),
            pltpu.VMEM((BLK_B, CH_B, M), bf16),
            pltpu.VMEM((BLK_B, CH_B, M), bf16),
            pltpu.VMEM((BLK_B, CH_B, M), bf16),
        ],
        compiler_params=pltpu.CompilerParams(
            dimension_semantics=("parallel", "arbitrary"),
            vmem_limit_bytes=100 * 1024 * 1024,
        ),
        name="memory_read",
    )(tk_all, zv_all, zrk_all, mem0)

    out = pl.pallas_call(
        _readout_body,
        out_shape=jax.ShapeDtypeStruct((S, B, M), f32),
        grid=(B // BLK_C, S),
        in_specs=[pl.BlockSpec((T, BLK_C, M), lambda b, c: (c, b, 0))],
        out_specs=pl.BlockSpec((1, BLK_C, M), lambda b, c: (c, b, 0)),
        scratch_shapes=[pltpu.VMEM((BLK_C, M), f32)] * 4,
        compiler_params=pltpu.CompilerParams(
            dimension_semantics=("parallel", "arbitrary"),
            vmem_limit_bytes=100 * 1024 * 1024,
        ),
        name="readout",
    )(rv_all)

    return out


# speedup vs baseline: 69.5437x; 1.1197x over previous
import functools

import jax
import jax.numpy as jnp
from jax import lax
from jax.experimental import pallas as pl
from jax.experimental.pallas import tpu as pltpu

S, B, I = 50, 256, 200
E, M = 80, 256
T = 50
T_TOT = S * T
A_I, A_V = 0.9, 0.9
ETA = 0.01
LAM = 0.951229424500714

WIN_LO, WIN_HI = 19, 48


def _spike(v):
    return 0.5 + 0.5 * jnp.tanh(5.0 * (v - 1.0))


def _dotT(x, w):
    return lax.dot_general(x, w, (((x.ndim - 1,), (1,)), ((), ())))


BLK_A = 64


def _chains_body(facts_ref, wemb_ref, wenc_ref, wcat_ref,
                 tk_out, zv_out, zrk_out,
                 ze_s, xin_s,
                 enc_i, enc_v, enc_z,
                 st_i, st_v, st_z, tk_s):
    c = pl.program_id(1)

    @pl.when(c == 0)
    def _():
        for r in (enc_i, enc_v, enc_z, st_i, st_v, st_z, tk_s):
            r[...] = jnp.zeros_like(r)

    x = facts_ref[0]
    emb = _dotT(x, wemb_ref[...])
    enc_in = _dotT(emb, wenc_ref[...])

    def p1(j, carry):
        i = A_I * enc_i[...] + enc_in
        v = A_V * enc_v[...] * (1.0 - enc_z[...]) + i
        z = _spike(v)
        enc_i[...] = i
        enc_v[...] = v
        enc_z[...] = z
        ze_s[pl.ds(j * BLK_A, BLK_A), :] = z
        return carry

    lax.fori_loop(0, T, p1, 0, unroll=2)

    xin_s[...] = _dotT(ze_s[...], wcat_ref[...])

    def p3(j, carry):
        sl = pl.ds(j * BLK_A, BLK_A)
        i = A_I * st_i[...] + xin_s[sl, :]
        v = A_V * st_v[...] * (1.0 - st_z[...]) + i
        z = _spike(v)
        st_i[...] = i
        st_v[...] = v
        st_z[...] = z

        kz = z[:, :M]
        tkn = LAM * tk_s[...] + kz
        tk_s[...] = tkn

        tk_out[j] = tkn.astype(jnp.bfloat16)
        zv_out[j] = z[:, M:2 * M].astype(jnp.bfloat16)
        zrk_out[j] = z[:, 2 * M:].astype(jnp.bfloat16)
        return carry

    lax.fori_loop(0, T, p3, 0, unroll=2)


BLK_B = 32
CH_B = 125


def _memory_body(tk_ref, zv_ref, zrk_ref, mem0_ref, rv_out,
                 mem_s, tkT, zvT, zrkT, rvT):
    c = pl.program_id(1)

    @pl.when(c == 0)
    def _():
        mem_s[...] = mem0_ref[...]

    tkT[...] = jnp.swapaxes(tk_ref[...], 0, 1)
    zvT[...] = jnp.swapaxes(zv_ref[...], 0, 1)
    zrkT[...] = jnp.swapaxes(zrk_ref[...], 0, 1)

    row = lax.broadcasted_iota(jnp.int32, (CH_B, CH_B), 0)
    col = lax.broadcasted_iota(jnp.int32, (CH_B, CH_B), 1)
    causal = col <= row

    for be in range(BLK_B):
        tkb = tkT[be]
        zvb = zvT[be]
        zrkb = zrkT[be]
        memb = mem_s[be]
        g = lax.dot_general(zrkb, tkb, (((1,), (1,)), ((), ())),
                            preferred_element_type=jnp.float32)
        gm = jnp.where(causal, g, 0.0).astype(jnp.bfloat16)
        intra = lax.dot_general(gm, zvb, (((1,), (0,)), ((), ())),
                                preferred_element_type=jnp.float32)
        inter = lax.dot_general(zrkb, memb.astype(jnp.bfloat16),
                                (((1,), (1,)), ((), ())),
                                preferred_element_type=jnp.float32)
        rvT[be] = (inter + ETA * intra).astype(jnp.bfloat16)
        mem_s[be] = memb + ETA * lax.dot_general(
            zvb, tkb, (((0,), (0,)), ((), ())),
            preferred_element_type=jnp.float32)

    rv_out[...] = jnp.swapaxes(rvT[...], 0, 1)


BLK_C = 128


def _readout_body(rv_ref, out_ref, ri_s, rv_s, rz_s, acc_s):
    c = pl.program_id(1)

    @pl.when(c == 0)
    def _():
        ri_s[...] = jnp.zeros_like(ri_s)
        rv_s[...] = jnp.zeros_like(rv_s)
        rz_s[...] = jnp.zeros_like(rz_s)

    acc_s[...] = jnp.zeros_like(acc_s)

    def step(j, carry, accumulate):
        i = A_I * ri_s[...] + rv_ref[j]
        v = A_V * rv_s[...] * (1.0 - rz_s[...]) + i
        z = _spike(v)
        ri_s[...] = i
        rv_s[...] = v
        rz_s[...] = z
        if accumulate:
            acc_s[...] = acc_s[...] + z
        return carry

    lax.fori_loop(0, WIN_LO, functools.partial(step, accumulate=False), 0,
                  unroll=2)
    lax.fori_loop(WIN_LO, WIN_HI + 1, functools.partial(step, accumulate=True),
                  0, unroll=2)
    lax.fori_loop(WIN_HI + 1, T, functools.partial(step, accumulate=False), 0,
                  unroll=2)

    out_ref[0] = acc_s[...]


def kernel(facts, mem0, W_emb, W_enc, W_wk, W_wv, W_rk):
    f32 = jnp.float32
    bf16 = jnp.bfloat16
    zseq = jax.ShapeDtypeStruct((T_TOT, B, M), bf16)
    W_cat = jnp.concatenate([W_wk, W_wv, W_rk], axis=0)

    tk_all, zv_all, zrk_all = pl.pallas_call(
        _chains_body,
        out_shape=(zseq, zseq, zseq),
        grid=(B // BLK_A, S),
        in_specs=[
            pl.BlockSpec((1, BLK_A, I), lambda b, c: (c, b, 0)),
            pl.BlockSpec((E, I), lambda b, c: (0, 0)),
            pl.BlockSpec((E, E), lambda b, c: (0, 0)),
            pl.BlockSpec((3 * M, E), lambda b, c: (0, 0)),
        ],
        out_specs=(
            pl.BlockSpec((T, BLK_A, M), lambda b, c: (c, b, 0)),
            pl.BlockSpec((T, BLK_A, M), lambda b, c: (c, b, 0)),
            pl.BlockSpec((T, BLK_A, M), lambda b, c: (c, b, 0)),
        ),
        scratch_shapes=[
            pltpu.VMEM((T * BLK_A, E), f32),
            pltpu.VMEM((T * BLK_A, 3 * M), f32),
        ] + [pltpu.VMEM((BLK_A, E), f32)] * 3
          + [pltpu.VMEM((BLK_A, 3 * M), f32)] * 3
          + [pltpu.VMEM((BLK_A, M), f32)],
        compiler_params=pltpu.CompilerParams(
            dimension_semantics=("parallel", "arbitrary"),
            vmem_limit_bytes=100 * 1024 * 1024,
        ),
        name="lif_chains",
    )(facts, W_emb, W_enc, W_cat)

    rv_all = pl.pallas_call(
        _memory_body,
        out_shape=zseq,
        grid=(B // BLK_B, T_TOT // CH_B),
        in_specs=[
            pl.BlockSpec((CH_B, BLK_B, M), lambda b, c: (c, b, 0)),
            pl.BlockSpec((CH_B, BLK_B, M), lambda b, c: (c, b, 0)),
            pl.BlockSpec((CH_B, BLK_B, M), lambda b, c: (c, b, 0)),
            pl.BlockSpec((BLK_B, M, M), lambda b, c: (b, 0, 0)),
        ],
        out_specs=pl.BlockSpec((CH_B, BLK_B, M), lambda b, c: (c, b, 0)),
        scratch_shapes=[
            pltpu.VMEM((BLK_B, M, M), f32),
            pltpu.VMEM((BLK_B, CH_B, M), bf16),
            pltpu.VMEM((BLK_B, CH_B, M), bf16),
            pltpu.VMEM((BLK_B, CH_B, M), bf16),
            pltpu.VMEM((BLK_B, CH_B, M), bf16),
        ],
        compiler_params=pltpu.CompilerParams(
            dimension_semantics=("parallel", "arbitrary"),
            vmem_limit_bytes=100 * 1024 * 1024,
        ),
        name="memory_read",
    )(tk_all, zv_all, zrk_all, mem0)

    out = pl.pallas_call(
        _readout_body,
        out_shape=jax.ShapeDtypeStruct((S, B, M), f32),
        grid=(B // BLK_C, S),
        in_specs=[pl.BlockSpec((T, BLK_C, M), lambda b, c: (c, b, 0))],
        out_specs=pl.BlockSpec((1, BLK_C, M), lambda b, c: (c, b, 0)),
        scratch_shapes=[pltpu.VMEM((BLK_C, M), f32)] * 4,
        compiler_params=pltpu.CompilerParams(
            dimension_semantics=("parallel", "arbitrary"),
            vmem_limit_bytes=100 * 1024 * 1024,
        ),
        name="readout",
    )(rv_all)

    return out
